# baseline (speedup 1.0000x reference)
import jax
import jax.numpy as jnp
from jax.experimental import pallas as pl

_N = 10000
_K_TAG = 3


def _matmul_pallas(x, w, b=None, relu=False):
    n, di = x.shape
    do = w.shape[1]
    blk = 1000

    def body(x_ref, w_ref, b_ref, o_ref):
        acc = jnp.dot(x_ref[...], w_ref[...], preferred_element_type=jnp.float32)
        acc = acc + b_ref[...]
        if relu:
            acc = jnp.maximum(acc, 0.0)
        o_ref[...] = acc

    bvec = jnp.zeros((1, do), jnp.float32) if b is None else b.reshape(1, do)
    return pl.pallas_call(
        body,
        grid=(n // blk,),
        in_specs=[
            pl.BlockSpec((blk, di), lambda i: (i, 0)),
            pl.BlockSpec((di, do), lambda i: (0, 0)),
            pl.BlockSpec((1, do), lambda i: (0, 0)),
        ],
        out_specs=pl.BlockSpec((blk, do), lambda i: (i, 0)),
        out_shape=jax.ShapeDtypeStruct((n, do), jnp.float32),
    )(x, w, bvec)


def _segment_softmax(scores, seg, num):
    m = jax.ops.segment_max(scores, seg, num_segments=num)
    m = jnp.where(jnp.isfinite(m), m, 0.0)
    e = jnp.exp(scores - m[seg])
    s = jax.ops.segment_sum(e, seg, num_segments=num)
    return e / (s[seg] + 1e-16)


def _gatv2(x, src, dst, p):
    n = x.shape[0]
    do = p['Wl'].shape[1]
    if do >= 128:
        xl = _matmul_pallas(x, p['Wl'])
        xr = _matmul_pallas(x, p['Wr'])
    else:
        xl = x @ p['Wl']
        xr = x @ p['Wr']
    h = jax.nn.leaky_relu(xl[src] + xr[dst], negative_slope=0.2)
    a = _segment_softmax(h @ p['att'], dst, n)
    return jax.ops.segment_sum(xl[src] * a[:, None], dst, num_segments=n) + p['b']


def _tag(x, src, dst, norm, p):
    n = x.shape[0]
    do = p['Ws'][0].shape[1]
    mm = (lambda a, w: _matmul_pallas(a, w)) if do >= 128 else (lambda a, w: a @ w)
    out = mm(x, p['Ws'][0])
    h = x
    for k in range(1, _K_TAG + 1):
        h = jax.ops.segment_sum(h[src] * norm[:, None], dst, num_segments=n)
        out = out + mm(h, p['Ws'][k])
    return out + p['b']


def _sage(x, src, dst, p):
    n = x.shape[0]
    do = p['Wl'].shape[1]
    s = jax.ops.segment_sum(x[src], dst, num_segments=n)
    cnt = jax.ops.segment_sum(jnp.ones((src.shape[0],), dtype=x.dtype), dst, num_segments=n)
    mean = s / jnp.maximum(cnt, 1.0)[:, None]
    if do >= 128:
        return _matmul_pallas(mean, p['Wl'], p['bl']) + _matmul_pallas(x, p['Wr'])
    return mean @ p['Wl'] + p['bl'] + x @ p['Wr']


def _combine_pallas(x1, x2, x3, y, mask, w, b):
    n = x1.shape[0]

    def body(x1_ref, x2_ref, x3_ref, y_ref, m_ref, w_ref, o_ref):
        out = x1_ref[...] * w_ref[0, 0] + x2_ref[...] * w_ref[1, 0] + x3_ref[...] * w_ref[2, 0] + w_ref[3, 0]
        out = jnp.maximum(out, 0.0)
        x_i = jnp.where(m_ref[...] != 0, out / 0.05, 0.0)
        o_ref[...] = jnp.where(y_ref[...] == 0.0, x_i, out)

    wb = jnp.concatenate([w.reshape(3, 1), b.reshape(1, 1)], axis=0)
    return pl.pallas_call(
        body,
        in_specs=[pl.BlockSpec((n, 1), lambda: (0, 0))] * 5
        + [pl.BlockSpec((4, 1), lambda: (0, 0))],
        out_specs=pl.BlockSpec((n, 1), lambda: (0, 0)),
        out_shape=jax.ShapeDtypeStruct((n, 1), jnp.float32),
    )(x1, x2, x3, y, mask, wb)


def kernel(x, edge_index, y, params):
    n = x.shape[0]
    src, dst = edge_index[0], edge_index[1]
    loop = jnp.arange(n, dtype=src.dtype)
    src_sl = jnp.concatenate([src, loop])
    dst_sl = jnp.concatenate([dst, loop])
    x1 = x
    for i, p in enumerate(params['gat']):
        x1 = _gatv2(x1, src_sl, dst_sl, p)
        if i != len(params['gat']) - 1:
            x1 = jax.nn.relu(x1)
    deg = jax.ops.segment_sum(jnp.ones((src.shape[0],), dtype=x.dtype), dst, num_segments=n)
    dis = jnp.where(deg > 0, 1.0 / jnp.sqrt(jnp.maximum(deg, 1e-12)), 0.0)
    norm = dis[src] * dis[dst]
    x2 = x
    for i, p in enumerate(params['tag']):
        x2 = _tag(x2, src, dst, norm, p)
        if i != len(params['tag']) - 1:
            x2 = jax.nn.relu(x2)
    x3 = x
    for i, p in enumerate(params['sage']):
        x3 = _sage(x3, src, dst, p)
        if i != len(params['sage']) - 1:
            x3 = jax.nn.relu(x3)
    keep = 1.0 - 0.95
    mask = jax.random.bernoulli(jax.random.key(42), keep, (n, 1)).astype(jnp.float32)
    return _combine_pallas(x1, x2, x3, y, mask, params['lin']['W'], params['lin']['b'])


# speedup vs baseline: 2.7305x; 2.7305x over previous
import functools

import jax
import jax.numpy as jnp
from jax import lax
from jax.experimental import pallas as pl
from jax.experimental.pallas import tpu as pltpu
from jax.experimental.pallas import tpu_sc as plsc

_N = 10000
_NACC = 10240
_TRASH = 10000
_C = 128
_NS = 16
_NC = 2
_MMBLK = 1000


def _mesh():
    return plsc.VectorSubcoreMesh(core_axis_name="c", subcore_axis_name="s")


def _writeout_rows(acc_sh, out_slice_fn, s):
    @pl.when(s < _NS - 1)
    def _():
        pltpu.sync_copy(acc_sh.at[pl.ds(s * 640, 640)], out_slice_fn(s * 640, 640))

    @pl.when(s == _NS - 1)
    def _():
        pltpu.sync_copy(acc_sh.at[pl.ds(9600, 400)], out_slice_fn(9600, 400))


def _sc_hop128(hh, src2, dst, w16=None):
    ep = dst.shape[0]
    k = ep // (_NS * _C)
    weighted = w16 is not None
    scratch = {
        "si_v": pltpu.VMEM((_C,), jnp.int32),
        "di_v": pltpu.VMEM((_C,), jnp.int32),
        "rows_v": pltpu.VMEM((_C, 128), jnp.float32),
        "acc_sh": pltpu.VMEM_SHARED((_NACC, 128), jnp.float32),
        "sem": pltpu.SemaphoreType.DMA,
    }
    if weighted:
        scratch["w_v"] = pltpu.VMEM((_C, 16), jnp.float32)

    def body(h_hbm, s2_hbm, d_hbm, w_hbm, z_hbm, out_hbm, *, si_v, di_v,
             rows_v, acc_sh, sem, w_v=None):
        c = lax.axis_index("c")
        s = lax.axis_index("s")
        zr = _NACC // _NS
        pltpu.sync_copy(z_hbm.at[pl.ds(s * zr, zr)], acc_sh.at[pl.ds(s * zr, zr)])
        plsc.subcore_barrier()

        def chunk(j, carry):
            off = s * (k * _C) + j * _C
            pltpu.sync_copy(s2_hbm.at[c, pl.ds(off, _C)], si_v)
            pltpu.sync_copy(d_hbm.at[pl.ds(off, _C)], di_v)
            pltpu.async_copy(h_hbm.at[si_v], rows_v, sem).wait()
            if weighted:
                pltpu.sync_copy(w_hbm.at[pl.ds(off, _C), :], w_v)

                def row(r, rc):
                    wv = w_v[r, :]
                    for u in range(8):
                        rows_v[r, pl.ds(u * 16, 16)] = rows_v[r, pl.ds(u * 16, 16)] * wv
                    return rc
                lax.fori_loop(0, _C, row, 0)
            pltpu.sync_copy(rows_v, acc_sh.at[di_v], add=True)
            return carry
        lax.fori_loop(0, k, chunk, 0)
        plsc.subcore_barrier()
        _writeout_rows(acc_sh, lambda r0, nr: out_hbm.at[c, pl.ds(r0, nr), :], s)

    def wrapped(*refs):
        names = list(scratch.keys())
        n_in = len(refs) - len(names)
        body(*refs[:n_in], **dict(zip(names, refs[n_in:])))

    zeros = jnp.zeros((_NACC, 128), jnp.float32)
    f = pl.kernel(
        wrapped,
        out_type=jax.ShapeDtypeStruct((2, _N, 128), jnp.float32),
        mesh=_mesh(),
        scratch_types=list(scratch.values()),
    )
    warg = w16 if weighted else jnp.zeros((1, 16), jnp.float32)
    return f(hh, src2, dst, warg, zeros)


def _sc_hop16(t16, src, dst, w16=None):
    ep = dst.shape[0]
    k = ep // (_NC * _NS * _C)
    weighted = w16 is not None
    scratch = {
        "si_v": pltpu.VMEM((_C,), jnp.int32),
        "di_v": pltpu.VMEM((_C,), jnp.int32),
        "rows_v": pltpu.VMEM((_C, 16), jnp.float32),
        "acc_sh": pltpu.VMEM_SHARED((_NACC, 16), jnp.float32),
        "sem": pltpu.SemaphoreType.DMA,
    }
    if weighted:
        scratch["w_v"] = pltpu.VMEM((_C, 16), jnp.float32)

    def body(t_hbm, s_hbm, d_hbm, w_hbm, z_hbm, out_hbm, *, si_v, di_v,
             rows_v, acc_sh, sem, w_v=None):
        c = lax.axis_index("c")
        s = lax.axis_index("s")
        zr = _NACC // _NS
        pltpu.sync_copy(z_hbm.at[pl.ds(s * zr, zr)], acc_sh.at[pl.ds(s * zr, zr)])
        plsc.subcore_barrier()

        def chunk(j, carry):
            off = (c * _NS + s) * (k * _C) + j * _C
            pltpu.sync_copy(s_hbm.at[pl.ds(off, _C)], si_v)
            pltpu.sync_copy(d_hbm.at[pl.ds(off, _C)], di_v)
            pltpu.async_copy(t_hbm.at[si_v], rows_v, sem).wait()
            if weighted:
                pltpu.sync_copy(w_hbm.at[pl.ds(off, _C), :], w_v)

                def row(r, rc):
                    rows_v[r, :] = rows_v[r, :] * w_v[r, :]
                    return rc
                lax.fori_loop(0, _C, row, 0)
            pltpu.sync_copy(rows_v, acc_sh.at[di_v], add=True)
            return carry
        lax.fori_loop(0, k, chunk, 0)
        plsc.subcore_barrier()
        _writeout_rows(acc_sh, lambda r0, nr: out_hbm.at[c, pl.ds(r0, nr), :], s)

    def wrapped(*refs):
        names = list(scratch.keys())
        n_in = len(refs) - len(names)
        body(*refs[:n_in], **dict(zip(names, refs[n_in:])))

    zeros = jnp.zeros((_NACC, 16), jnp.float32)
    f = pl.kernel(
        wrapped,
        out_type=jax.ShapeDtypeStruct((2, _N, 16), jnp.float32),
        mesh=_mesh(),
        scratch_types=list(scratch.values()),
        compiler_params=pltpu.CompilerParams(use_tc_tiling_on_sc=False),
    )
    warg = w16 if weighted else jnp.zeros((1, 16), jnp.float32)
    return f(t16, src, dst, warg, zeros)


def _sc_scatter16(vals16, dst):
    ep = dst.shape[0]
    k = ep // (_NC * _NS * _C)
    scratch = [
        pltpu.VMEM((_C,), jnp.int32),
        pltpu.VMEM((_C, 16), jnp.float32),
        pltpu.VMEM_SHARED((_NACC, 16), jnp.float32),
    ]

    def body(v_hbm, d_hbm, z_hbm, out_hbm, di_v, rows_v, acc_sh):
        c = lax.axis_index("c")
        s = lax.axis_index("s")
        zr = _NACC // _NS
        pltpu.sync_copy(z_hbm.at[pl.ds(s * zr, zr)], acc_sh.at[pl.ds(s * zr, zr)])
        plsc.subcore_barrier()

        def chunk(j, carry):
            off = (c * _NS + s) * (k * _C) + j * _C
            pltpu.sync_copy(d_hbm.at[pl.ds(off, _C)], di_v)
            pltpu.sync_copy(v_hbm.at[pl.ds(off, _C), :], rows_v)
            pltpu.sync_copy(rows_v, acc_sh.at[di_v], add=True)
            return carry
        lax.fori_loop(0, k, chunk, 0)
        plsc.subcore_barrier()
        _writeout_rows(acc_sh, lambda r0, nr: out_hbm.at[c, pl.ds(r0, nr), :], s)

    zeros = jnp.zeros((_NACC, 16), jnp.float32)
    f = pl.kernel(
        body,
        out_type=jax.ShapeDtypeStruct((2, _N, 16), jnp.float32),
        mesh=_mesh(),
        scratch_types=scratch,
    )
    return f(vals16, dst, zeros)


def _sc_gather_pair128(ta, tb, ia2, ib2):
    ep = ia2.shape[1]
    k = ep // (_NS * _C)
    scratch = [
        pltpu.VMEM((_C,), jnp.int32),
        pltpu.VMEM((_C,), jnp.int32),
        pltpu.VMEM((_C, 128), jnp.float32),
        pltpu.VMEM((_C, 128), jnp.float32),
        pltpu.SemaphoreType.DMA,
        pltpu.SemaphoreType.DMA,
    ]

    def body(ta_hbm, tb_hbm, ia_hbm, ib_hbm, oa_hbm, ob_hbm,
             ia_v, ib_v, ra_v, rb_v, sema, semb):
        c = lax.axis_index("c")
        s = lax.axis_index("s")

        def chunk(j, carry):
            off = s * (k * _C) + j * _C
            pltpu.sync_copy(ia_hbm.at[c, pl.ds(off, _C)], ia_v)
            pltpu.sync_copy(ib_hbm.at[c, pl.ds(off, _C)], ib_v)
            da = pltpu.async_copy(ta_hbm.at[ia_v], ra_v, sema)
            db = pltpu.async_copy(tb_hbm.at[ib_v], rb_v, semb)
            da.wait()
            db.wait()
            pltpu.sync_copy(ra_v, oa_hbm.at[c, pl.ds(off, _C), :])
            pltpu.sync_copy(rb_v, ob_hbm.at[c, pl.ds(off, _C), :])
            return carry
        lax.fori_loop(0, k, chunk, 0)

    f = pl.kernel(
        body,
        out_type=(jax.ShapeDtypeStruct((2, ep, 128), jnp.float32),
                  jax.ShapeDtypeStruct((2, ep, 128), jnp.float32)),
        mesh=_mesh(),
        scratch_types=scratch,
    )
    return f(ta, tb, ia2, ib2)


def _sc_gather_pair16(t16, ia, ib):
    ep = ia.shape[0]
    k = ep // (_NC * _NS * _C)
    scratch = [
        pltpu.VMEM((_C,), jnp.int32),
        pltpu.VMEM((_C,), jnp.int32),
        pltpu.VMEM((_C, 16), jnp.float32),
        pltpu.VMEM((_C, 16), jnp.float32),
        pltpu.SemaphoreType.DMA,
        pltpu.SemaphoreType.DMA,
    ]

    def body(t_hbm, ia_hbm, ib_hbm, oa_hbm, ob_hbm,
             ia_v, ib_v, ra_v, rb_v, sema, semb):
        c = lax.axis_index("c")
        s = lax.axis_index("s")

        def chunk(j, carry):
            off = (c * _NS + s) * (k * _C) + j * _C
            pltpu.sync_copy(ia_hbm.at[pl.ds(off, _C)], ia_v)
            pltpu.sync_copy(ib_hbm.at[pl.ds(off, _C)], ib_v)
            da = pltpu.async_copy(t_hbm.at[ia_v], ra_v, sema)
            db = pltpu.async_copy(t_hbm.at[ib_v], rb_v, semb)
            da.wait()
            db.wait()
            pltpu.sync_copy(ra_v, oa_hbm.at[pl.ds(off, _C), :])
            pltpu.sync_copy(rb_v, ob_hbm.at[pl.ds(off, _C), :])
            return carry
        lax.fori_loop(0, k, chunk, 0)

    f = pl.kernel(
        body,
        out_type=(jax.ShapeDtypeStruct((ep, 16), jnp.float32),
                  jax.ShapeDtypeStruct((ep, 16), jnp.float32)),
        mesh=_mesh(),
        scratch_types=scratch,
        compiler_params=pltpu.CompilerParams(use_tc_tiling_on_sc=False),
    )
    return f(t16, ia, ib)


def _mm_h(xh, w, bias=None, relu=False, acc=None, row_scale=None):
    nb = _N // _MMBLK
    has_b = bias is not None
    has_a = acc is not None
    has_s = row_scale is not None

    def body(*refs):
        i = 0
        x_ref = refs[i]; i += 1
        w_ref = refs[i]; i += 1
        s_ref = refs[i] if has_s else None
        i += has_s
        b_ref = refs[i] if has_b else None
        i += has_b
        a_ref = refs[i] if has_a else None
        i += has_a
        o_ref = refs[i]
        ci = pl.program_id(2)
        xv = x_ref[...]
        if has_s:
            xv = xv * s_ref[...]
        contrib = jnp.dot(xv, w_ref[...], preferred_element_type=jnp.float32)

        @pl.when(ci == 0)
        def _():
            r = contrib
            if has_b:
                r = r + b_ref[...]
            if has_a:
                r = r + a_ref[...]
            o_ref[...] = r

        @pl.when(ci == 1)
        def _():
            r = o_ref[...] + contrib
            if relu:
                r = jnp.maximum(r, 0.0)
            o_ref[...] = r

    in_specs = [
        pl.BlockSpec((_MMBLK, 128), lambda i, co, ci: (ci * nb + i, 0)),
        pl.BlockSpec((128, 128), lambda i, co, ci: (ci, co)),
    ]
    args = [xh, w]
    if has_s:
        in_specs.append(pl.BlockSpec((_MMBLK, 1), lambda i, co, ci: (ci * nb + i, 0)))
        args.append(row_scale)
    if has_b:
        in_specs.append(pl.BlockSpec((1, 128), lambda i, co, ci: (0, co)))
        args.append(bias.reshape(1, 256))
    if has_a:
        in_specs.append(pl.BlockSpec((_MMBLK, 128), lambda i, co, ci: (co * nb + i, 0)))
        args.append(acc)
    return pl.pallas_call(
        body,
        grid=(nb, 2, 2),
        in_specs=in_specs,
        out_specs=pl.BlockSpec((_MMBLK, 128), lambda i, co, ci: (co * nb + i, 0)),
        out_shape=jax.ShapeDtypeStruct((2 * _N, 128), jnp.float32),
    )(*args)


def _mm_thin(xh, w16):
    nb = _N // _MMBLK

    def body(x0_ref, x1_ref, w_ref, o_ref):
        o_ref[...] = (
            jnp.dot(x0_ref[...], w_ref[0:128, :], preferred_element_type=jnp.float32)
            + jnp.dot(x1_ref[...], w_ref[128:256, :], preferred_element_type=jnp.float32))

    return pl.pallas_call(
        body,
        grid=(nb,),
        in_specs=[
            pl.BlockSpec((_MMBLK, 128), lambda i: (i, 0)),
            pl.BlockSpec((_MMBLK, 128), lambda i: (nb + i, 0)),
            pl.BlockSpec((256, 16), lambda i: (0, 0)),
        ],
        out_specs=pl.BlockSpec((_MMBLK, 16), lambda i: (i, 0)),
        out_shape=jax.ShapeDtypeStruct((_N, 16), jnp.float32),
    )(xh, xh, w16)


_EBLK = 1024


def _tc_score128(gl, gr, att):
    ep = gl.shape[1]
    ne = ep // _EBLK

    def body(gl0, gl1, gr0, gr1, att_ref, s_ref, m_ref):
        i = pl.program_id(0)
        t0 = gl0[0] + gr0[0]
        t1 = gl1[0] + gr1[0]
        t0 = jnp.where(t0 >= 0, t0, 0.2 * t0)
        t1 = jnp.where(t1 >= 0, t1, 0.2 * t1)
        s = (jnp.sum(t0 * att_ref[0:1, :], axis=-1, keepdims=True)
             + jnp.sum(t1 * att_ref[1:2, :], axis=-1, keepdims=True))
        s_ref[...] = s
        bm = jnp.max(s, keepdims=True)

        @pl.when(i == 0)
        def _():
            m_ref[...] = bm

        @pl.when(i > 0)
        def _():
            m_ref[...] = jnp.maximum(m_ref[...], bm)

    return pl.pallas_call(
        body,
        grid=(ne,),
        in_specs=[
            pl.BlockSpec((1, _EBLK, 128), lambda i: (0, i, 0)),
            pl.BlockSpec((1, _EBLK, 128), lambda i: (1, i, 0)),
            pl.BlockSpec((1, _EBLK, 128), lambda i: (0, i, 0)),
            pl.BlockSpec((1, _EBLK, 128), lambda i: (1, i, 0)),
            pl.BlockSpec((2, 128), lambda i: (0, 0)),
        ],
        out_specs=[
            pl.BlockSpec((_EBLK, 1), lambda i: (i, 0)),
            pl.BlockSpec((1, 1), lambda i: (0, 0)),
        ],
        out_shape=[
            jax.ShapeDtypeStruct((ep, 1), jnp.float32),
            jax.ShapeDtypeStruct((1, 1), jnp.float32),
        ],
    )(gl, gl, gr, gr, att.reshape(2, 128))


def _tc_score16(g1, g2, att0):
    ep = g1.shape[0]
    ne = ep // _EBLK

    def body(g1_ref, g2_ref, a_ref, s_ref, m_ref):
        i = pl.program_id(0)
        t = g1_ref[:, 0:1] + g2_ref[:, 1:2]
        t = jnp.where(t >= 0, t, 0.2 * t)
        s = t * a_ref[0, 0]
        s_ref[...] = s
        bm = jnp.max(s, keepdims=True)

        @pl.when(i == 0)
        def _():
            m_ref[...] = bm

        @pl.when(i > 0)
        def _():
            m_ref[...] = jnp.maximum(m_ref[...], bm)

    return pl.pallas_call(
        body,
        grid=(ne,),
        in_specs=[
            pl.BlockSpec((_EBLK, 16), lambda i: (i, 0)),
            pl.BlockSpec((_EBLK, 16), lambda i: (i, 0)),
            pl.BlockSpec((1, 1), lambda i: (0, 0)),
        ],
        out_specs=[
            pl.BlockSpec((_EBLK, 1), lambda i: (i, 0)),
            pl.BlockSpec((1, 1), lambda i: (0, 0)),
        ],
        out_shape=[
            jax.ShapeDtypeStruct((ep, 1), jnp.float32),
            jax.ShapeDtypeStruct((1, 1), jnp.float32),
        ],
    )(g1, g2, att0.reshape(1, 1))


def _tc_exp16(s, gmax):
    ep = s.shape[0]
    ne = ep // _EBLK

    def body(s_ref, m_ref, o_ref):
        e = jnp.exp(s_ref[...] - m_ref[0, 0])
        o_ref[...] = jnp.broadcast_to(e, (_EBLK, 16))

    return pl.pallas_call(
        body,
        grid=(ne,),
        in_specs=[
            pl.BlockSpec((_EBLK, 1), lambda i: (i, 0)),
            pl.BlockSpec((1, 1), lambda i: (0, 0)),
        ],
        out_specs=pl.BlockSpec((_EBLK, 16), lambda i: (i, 0)),
        out_shape=jax.ShapeDtypeStruct((ep, 16), jnp.float32),
    )(s, gmax)


def _tc_rowscale(a, s2, bias=None, relu=False, div=False):
    nb = _N // _MMBLK
    has_b = bias is not None

    def body(*refs):
        a_ref, s_ref = refs[0], refs[1]
        b_ref = refs[2] if has_b else None
        o_ref = refs[-1]
        v = a_ref[...] / s_ref[...] if div else a_ref[...] * s_ref[...]
        if has_b:
            v = v + b_ref[...]
        if relu:
            v = jnp.maximum(v, 0.0)
        o_ref[...] = v

    in_specs = [
        pl.BlockSpec((_MMBLK, 128), lambda i: (i, 0)),
        pl.BlockSpec((_MMBLK, 1), lambda i: (i, 0)),
    ]
    args = [a, s2]
    if has_b:
        in_specs.append(pl.BlockSpec((1, 128), lambda i: (0, i // nb)))
        args.append(bias.reshape(1, 256))
    return pl.pallas_call(
        body,
        grid=(2 * nb,),
        in_specs=in_specs,
        out_specs=pl.BlockSpec((_MMBLK, 128), lambda i: (i, 0)),
        out_shape=jax.ShapeDtypeStruct((2 * _N, 128), jnp.float32),
    )(*args)


def _tc_scale16(a, s):
    nb = _N // _MMBLK

    def body(a_ref, s_ref, o_ref):
        o_ref[...] = a_ref[...] * s_ref[...]

    return pl.pallas_call(
        body,
        grid=(nb,),
        in_specs=[
            pl.BlockSpec((_MMBLK, 16), lambda i: (i, 0)),
            pl.BlockSpec((_MMBLK, 1), lambda i: (i, 0)),
        ],
        out_specs=pl.BlockSpec((_MMBLK, 16), lambda i: (i, 0)),
        out_shape=jax.ShapeDtypeStruct((_N, 16), jnp.float32),
    )(a, s)


def _tc_scale16p(p, s):
    nb = _N // _MMBLK

    def body(p_ref, s_ref, o_ref):
        o_ref[...] = (p_ref[0] + p_ref[1]) * s_ref[...]

    return pl.pallas_call(
        body,
        grid=(nb,),
        in_specs=[
            pl.BlockSpec((2, _MMBLK, 16), lambda i: (0, i, 0)),
            pl.BlockSpec((_MMBLK, 1), lambda i: (i, 0)),
        ],
        out_specs=pl.BlockSpec((_MMBLK, 16), lambda i: (i, 0)),
        out_shape=jax.ShapeDtypeStruct((_N, 16), jnp.float32),
    )(p, s)


def _tc_prep(degp):
    nb = _N // _MMBLK

    def body(d_ref, dis_ref, ic_ref):
        deg = d_ref[0, :, 0:1] + d_ref[1, :, 0:1]
        dis = jnp.where(deg > 0, jax.lax.rsqrt(jnp.maximum(deg, 1e-12)), 0.0)
        dis_ref[...] = dis
        ic_ref[...] = 1.0 / jnp.maximum(deg, 1.0)

    return pl.pallas_call(
        body,
        grid=(nb,),
        in_specs=[pl.BlockSpec((2, _MMBLK, 16), lambda i: (0, i, 0))],
        out_specs=[
            pl.BlockSpec((_MMBLK, 1), lambda i: (i, 0)),
            pl.BlockSpec((_MMBLK, 1), lambda i: (i, 0)),
        ],
        out_shape=[
            jax.ShapeDtypeStruct((_N, 1), jnp.float32),
            jax.ShapeDtypeStruct((_N, 1), jnp.float32),
        ],
    )(degp)


def _tc_final(n4p, z4p, u_tag, p1, p2, p3, v_sage, hs_p, invcnt, scal, y, mask):
    nb = _N // _MMBLK

    def body(n4, z4, ut, p1r, p2r, p3r, vs, hs, ic, sc, y_ref, m_ref, o_ref):
        b4 = sc[0, 0]
        btag = sc[1, 0]
        bl3 = sc[2, 0]
        w0, w1, w2, blin = sc[3, 0], sc[4, 0], sc[5, 0], sc[6, 0]
        x1 = (n4[0, :, 0:1] + n4[1, :, 0:1]) / (z4[0, :, 0:1] + z4[1, :, 0:1]) + b4
        x2 = ut[:, 3:4] + p1r[:, 0:1] + p2r[:, 1:2] + p3r[:, 2:3] + btag
        x3 = (hs[0, :, 0:1] + hs[1, :, 0:1]) * ic[...] + bl3 + vs[:, 1:2]
        out = jnp.maximum(x1 * w0 + x2 * w1 + x3 * w2 + blin, 0.0)
        x_i = jnp.where(m_ref[...] != 0, out / 0.05, 0.0)
        o_ref[...] = jnp.where(y_ref[...] == 0.0, x_i, out)

    blk2 = pl.BlockSpec((2, _MMBLK, 16), lambda i: (0, i, 0))
    blk16 = pl.BlockSpec((_MMBLK, 16), lambda i: (i, 0))
    blk1 = pl.BlockSpec((_MMBLK, 1), lambda i: (i, 0))
    return pl.pallas_call(
        body,
        grid=(nb,),
        in_specs=[blk2, blk2, blk16, blk16, blk16, blk16, blk16, blk2,
                  blk1, pl.BlockSpec((7, 1), lambda i: (0, 0)), blk1, blk1],
        out_specs=blk1,
        out_shape=jax.ShapeDtypeStruct((_N, 1), jnp.float32),
    )(n4p, z4p, u_tag, p1, p2, p3, v_sage, hs_p, invcnt, scal, y, mask)


def _pad1(a, ep, fill):
    return jnp.concatenate(
        [a.astype(jnp.int32),
         jnp.full((ep - a.shape[0],), fill, jnp.int32)])


def kernel(x, edge_index, y, params):
    n = _N
    src = edge_index[0].astype(jnp.int32)
    dst = edge_index[1].astype(jnp.int32)
    e = src.shape[0]
    loop = jnp.arange(n, dtype=jnp.int32)

    ep1 = ((e + 4095) // 4096) * 4096
    ep2 = ((e + n + 4095) // 4096) * 4096

    srcp1 = _pad1(src, ep1, 0)
    dstp1 = _pad1(dst, ep1, _TRASH)
    src2_1 = jnp.stack([srcp1, srcp1 + n])

    src_sl = jnp.concatenate([src, loop])
    dst_sl = jnp.concatenate([dst, loop])
    srcp2 = _pad1(src_sl, ep2, 0)
    dstp2 = _pad1(dst_sl, ep2, _TRASH)
    src2_2 = jnp.stack([srcp2, srcp2 + n])
    dst2_2 = jnp.stack([dstp2, dstp2 + n])

    xh = jnp.transpose(x.reshape(n, 2, 128), (1, 0, 2)).reshape(2 * n, 128)

    ones16 = jnp.ones((ep1, 16), jnp.float32)
    degp = _sc_scatter16(ones16, dstp1)
    dis, invcnt = _tc_prep(degp)
    dis2 = jnp.concatenate([dis, dis], axis=0)
    dis2sq = dis2 * dis2
    invcnt2 = jnp.concatenate([invcnt, invcnt], axis=0)

    x1h = xh
    for p in params['gat'][:3]:
        xl = _mm_h(x1h, p['Wl'])
        xr = _mm_h(x1h, p['Wr'])
        gl, gr = _sc_gather_pair128(xl, xr, src2_2, dst2_2)
        s, gmax = _tc_score128(gl, gr, p['att'])
        e16 = _tc_exp16(s, gmax)
        zp = _sc_scatter16(e16, dstp2)
        z = zp[0, :, 0:1] + zp[1, :, 0:1]
        z2 = jnp.concatenate([z, z], axis=0)
        numer = _sc_hop128(xl, src2_2, dstp2, w16=e16).reshape(2 * n, 128)
        x1h = _tc_rowscale(numer, z2, bias=p['b'], relu=True, div=True)

    p4 = params['gat'][3]
    w4 = jnp.concatenate(
        [p4['Wl'], p4['Wr'], jnp.zeros((256, 14), jnp.float32)], axis=1)
    t4 = _mm_thin(x1h, w4)
    g1, g2 = _sc_gather_pair16(t4, srcp2, dstp2)
    s4, gmax4 = _tc_score16(g1, g2, p4['att'])
    e4 = _tc_exp16(s4, gmax4)
    z4p = _sc_scatter16(e4, dstp2)
    n4p = _sc_hop16(t4, srcp2, dstp2, w16=e4)

    x2h = xh
    for li, p in enumerate(params['tag'][:2]):
        out = _mm_h(x2h, p['Ws'][0])
        hs = _tc_rowscale(x2h, dis2)
        for kk in range(1, 4):
            raw = _sc_hop128(hs, src2_1, dstp1).reshape(2 * n, 128)
            last = kk == 3
            out = _mm_h(raw, p['Ws'][kk], row_scale=dis2, acc=out,
                        bias=p['b'] if last else None, relu=last)
            if not last:
                hs = _tc_rowscale(raw, dis2sq)
        x2h = out

    p3t = params['tag'][2]
    w16t = jnp.concatenate(
        [p3t['Ws'][1], p3t['Ws'][2], p3t['Ws'][3], p3t['Ws'][0],
         jnp.zeros((256, 12), jnp.float32)], axis=1)
    u_tag = _mm_thin(x2h, w16t)
    q = _tc_scale16(u_tag, dis)
    h1 = _sc_hop16(q, srcp1, dstp1)
    pp1 = _tc_scale16p(h1, dis)
    q = _tc_scale16(pp1, dis)
    h2 = _sc_hop16(q, srcp1, dstp1)
    pp2 = _tc_scale16p(h2, dis)
    q = _tc_scale16(pp2, dis)
    h3 = _sc_hop16(q, srcp1, dstp1)
    pp3 = _tc_scale16p(h3, dis)

    x3h = xh
    for p in params['sage'][:2]:
        raw = _sc_hop128(x3h, src2_1, dstp1).reshape(2 * n, 128)
        out = _mm_h(raw, p['Wl'], row_scale=invcnt2, bias=p['bl'])
        x3h = _mm_h(x3h, p['Wr'], acc=out, relu=True)

    p3s = params['sage'][2]
    w16s = jnp.concatenate(
        [p3s['Wl'], p3s['Wr'], jnp.zeros((256, 14), jnp.float32)], axis=1)
    v_sage = _mm_thin(x3h, w16s)
    hs_p = _sc_hop16(v_sage, srcp1, dstp1)

    scal = jnp.stack([
        p4['b'][0], p3t['b'][0], p3s['bl'][0],
        params['lin']['W'][0, 0], params['lin']['W'][1, 0],
        params['lin']['W'][2, 0], params['lin']['b'][0],
    ]).reshape(7, 1)
    mask = jax.random.bernoulli(jax.random.key(42), 0.05, (n, 1)).astype(jnp.float32)
    return _tc_final(n4p, z4p, u_tag, pp1, pp2, pp3, v_sage, hs_p, invcnt,
                     scal, y, mask)
